# baseline (device time: 200767 ns/iter reference)
import jax
import jax.numpy as jnp
from jax import lax
from jax.experimental import pallas as pl
from jax.experimental.pallas import tpu as pltpu


def kernel(x, pi):
    def body(pi_ref, x_ref, out_ref, send_sem, recv_sem):
        my_x = lax.axis_index("x")
        my_y = lax.axis_index("y")
        my_z = lax.axis_index("z")
        dst_y = pi_ref[my_y]

        rdma = pltpu.make_async_remote_copy(
            src_ref=x_ref,
            dst_ref=out_ref,
            send_sem=send_sem,
            recv_sem=recv_sem,
            device_id=(my_x, dst_y, my_z),
            device_id_type=pl.DeviceIdType.MESH,
        )
        rdma.start()
        rdma.wait()

    return pl.pallas_call(
        body,
        out_shape=jax.ShapeDtypeStruct(x.shape, jnp.float32),
        in_specs=[
            pl.BlockSpec(memory_space=pltpu.SMEM),
            pl.BlockSpec(memory_space=pltpu.VMEM),
        ],
        out_specs=pl.BlockSpec(memory_space=pltpu.VMEM),
        scratch_shapes=[
            pltpu.SemaphoreType.DMA,
            pltpu.SemaphoreType.DMA,
        ],
    )(pi, x)


# device time: 191749 ns/iter; 1.0470x vs baseline; 1.0470x over previous
import jax
import jax.numpy as jnp
from jax import lax
from jax.experimental import pallas as pl
from jax.experimental.pallas import tpu as pltpu


def kernel(x, pi):
    def body(pi_ref, x_ref, out_ref, send_sem, recv_sem):
        my_x = lax.axis_index("x")
        my_y = lax.axis_index("y")
        my_z = lax.axis_index("z")
        dst_y = pi_ref[my_y]
        src_y = jnp.int32(0)
        for j in range(4):
            src_y = jnp.where(pi_ref[j] == my_y, jnp.int32(j), src_y)

        barrier = pltpu.get_barrier_semaphore()
        pl.semaphore_signal(
            barrier,
            inc=1,
            device_id=(my_x, src_y, my_z),
            device_id_type=pl.DeviceIdType.MESH,
        )
        pl.semaphore_wait(barrier, 1)

        rdma = pltpu.make_async_remote_copy(
            src_ref=x_ref,
            dst_ref=out_ref,
            send_sem=send_sem,
            recv_sem=recv_sem,
            device_id=(my_x, dst_y, my_z),
            device_id_type=pl.DeviceIdType.MESH,
        )
        rdma.start()
        rdma.wait()

    return pl.pallas_call(
        body,
        out_shape=jax.ShapeDtypeStruct(x.shape, jnp.float32),
        in_specs=[
            pl.BlockSpec(memory_space=pltpu.SMEM),
            pl.BlockSpec(memory_space=pl.ANY),
        ],
        out_specs=pl.BlockSpec(memory_space=pl.ANY),
        scratch_shapes=[
            pltpu.SemaphoreType.DMA,
            pltpu.SemaphoreType.DMA,
        ],
        compiler_params=pltpu.CompilerParams(collective_id=0),
    )(pi, x)


# device time: 88767 ns/iter; 2.2617x vs baseline; 2.1601x over previous
import jax
import jax.numpy as jnp
from jax import lax
from jax.experimental import pallas as pl
from jax.experimental.pallas import tpu as pltpu

N_Y = 4
N_RING = 8
ROWS = 2048
PART = ROWS // N_RING


def kernel(x, pi):
    def body(pi_ref, x_ref, out_ref, send_sems, recv_sems):
        my_x = lax.axis_index("x")
        my_y = lax.axis_index("y")
        my_z = lax.axis_index("z")
        dst_y = pi_ref[my_y]
        src_y = jnp.int32(0)
        for j in range(N_Y):
            src_y = jnp.where(pi_ref[j] == my_y, jnp.int32(j), src_y)

        my_r = jnp.where(my_x == 0, my_z, 7 - my_z).astype(jnp.int32)

        def ring_coords(r):
            rx = jnp.where(r >= 4, 1, 0).astype(jnp.int32)
            rz = jnp.where(r >= 4, 7 - r, r).astype(jnp.int32)
            return rx, rz

        cw_x, cw_z = ring_coords((my_r + 1) % N_RING)
        ccw_x, ccw_z = ring_coords((my_r - 1) % N_RING)

        def blk(ref, p):
            return ref.at[:, pl.ds(p * PART, PART), :]

        def wait_part(p, slot):
            pltpu.make_async_remote_copy(
                src_ref=blk(out_ref, p),
                dst_ref=blk(out_ref, p),
                send_sem=send_sems.at[0],
                recv_sem=recv_sems.at[slot],
                device_id=(0, 0, 0),
                device_id_type=pl.DeviceIdType.MESH,
            ).wait_recv()

        barrier = pltpu.get_barrier_semaphore()
        for dev in [
            (my_x, src_y, my_z),
            (ccw_x, my_y, ccw_z),
            (cw_x, my_y, cw_z),
        ]:
            pl.semaphore_signal(
                barrier, inc=1, device_id=dev,
                device_id_type=pl.DeviceIdType.MESH,
            )
        pl.semaphore_wait(barrier, 3)

        sends = []

        p1 = pltpu.make_async_remote_copy(
            src_ref=blk(x_ref, my_r),
            dst_ref=blk(out_ref, my_r),
            send_sem=send_sems.at[0],
            recv_sem=recv_sems.at[7],
            device_id=(my_x, dst_y, my_z),
            device_id_type=pl.DeviceIdType.MESH,
        )
        p1.start()
        sends.append(p1)

        wait_part(my_r, 7)
        for slot, (tx, ty, tz), rslot in [
            (1, (cw_x, my_y, cw_z), 0),
            (5, (ccw_x, my_y, ccw_z), 4),
        ]:
            d = pltpu.make_async_remote_copy(
                src_ref=blk(out_ref, my_r),
                dst_ref=blk(out_ref, my_r),
                send_sem=send_sems.at[slot],
                recv_sem=recv_sems.at[rslot],
                device_id=(tx, ty, tz),
                device_id_type=pl.DeviceIdType.MESH,
            )
            d.start()
            sends.append(d)

        for s in range(1, 4):
            part_cw = (my_r - s) % N_RING
            wait_part(part_cw, s - 1)
            d = pltpu.make_async_remote_copy(
                src_ref=blk(out_ref, part_cw),
                dst_ref=blk(out_ref, part_cw),
                send_sem=send_sems.at[1 + s],
                recv_sem=recv_sems.at[s],
                device_id=(cw_x, my_y, cw_z),
                device_id_type=pl.DeviceIdType.MESH,
            )
            d.start()
            sends.append(d)
            if s <= 2:
                part_ccw = (my_r + s) % N_RING
                wait_part(part_ccw, 4 + s - 1)
                d = pltpu.make_async_remote_copy(
                    src_ref=blk(out_ref, part_ccw),
                    dst_ref=blk(out_ref, part_ccw),
                    send_sem=send_sems.at[5 + s],
                    recv_sem=recv_sems.at[4 + s],
                    device_id=(ccw_x, my_y, ccw_z),
                    device_id_type=pl.DeviceIdType.MESH,
                )
                d.start()
                sends.append(d)

        wait_part((my_r - 4) % N_RING, 3)
        wait_part((my_r + 3) % N_RING, 6)

        for d in sends:
            d.wait_send()

    return pl.pallas_call(
        body,
        out_shape=jax.ShapeDtypeStruct(x.shape, jnp.float32),
        in_specs=[
            pl.BlockSpec(memory_space=pltpu.SMEM),
            pl.BlockSpec(memory_space=pl.ANY),
        ],
        out_specs=pl.BlockSpec(memory_space=pltpu.VMEM),
        scratch_shapes=[
            pltpu.SemaphoreType.DMA((8,)),
            pltpu.SemaphoreType.DMA((8,)),
        ],
        compiler_params=pltpu.CompilerParams(collective_id=0),
    )(pi, x)


# device time: 75292 ns/iter; 2.6665x vs baseline; 1.1790x over previous
import jax
import jax.numpy as jnp
from jax import lax
from jax.experimental import pallas as pl
from jax.experimental.pallas import tpu as pltpu

N_Y = 4
N_RING = 8
ROWS = 2048
PART = ROWS // N_RING
K = 4
CHUNK = PART // K


def kernel(x, pi):
    def body(pi_ref, x_ref, out_ref, send_sems, recv_sems):
        my_x = lax.axis_index("x")
        my_y = lax.axis_index("y")
        my_z = lax.axis_index("z")
        dst_y = pi_ref[my_y]
        src_y = jnp.int32(0)
        for j in range(N_Y):
            src_y = jnp.where(pi_ref[j] == my_y, jnp.int32(j), src_y)

        my_r = jnp.where(my_x == 0, my_z, 7 - my_z).astype(jnp.int32)

        def ring_coords(r):
            rx = jnp.where(r >= 4, 1, 0).astype(jnp.int32)
            rz = jnp.where(r >= 4, 7 - r, r).astype(jnp.int32)
            return rx, rz

        cw_x, cw_z = ring_coords((my_r + 1) % N_RING)
        ccw_x, ccw_z = ring_coords((my_r - 1) % N_RING)

        def blk(ref, p, c):
            return ref.at[:, pl.ds(p * PART + c * CHUNK, CHUNK), :]

        def wait_chunk(p, stage, c):
            pltpu.make_async_remote_copy(
                src_ref=blk(out_ref, p, c),
                dst_ref=blk(out_ref, p, c),
                send_sem=send_sems.at[0],
                recv_sem=recv_sems.at[stage * K + c],
                device_id=(0, 0, 0),
                device_id_type=pl.DeviceIdType.MESH,
            ).wait_recv()

        sends = []

        def send_chunk(src_ref, p, c, send_stage, recv_stage, dev):
            d = pltpu.make_async_remote_copy(
                src_ref=blk(src_ref, p, c),
                dst_ref=blk(out_ref, p, c),
                send_sem=send_sems.at[send_stage * K + c],
                recv_sem=recv_sems.at[recv_stage * K + c],
                device_id=dev,
                device_id_type=pl.DeviceIdType.MESH,
            )
            d.start()
            sends.append(d)

        barrier = pltpu.get_barrier_semaphore()
        for dev in [
            (my_x, src_y, my_z),
            (ccw_x, my_y, ccw_z),
            (cw_x, my_y, cw_z),
        ]:
            pl.semaphore_signal(
                barrier, inc=1, device_id=dev,
                device_id_type=pl.DeviceIdType.MESH,
            )
        pl.semaphore_wait(barrier, 3)

        cw_dev = (cw_x, my_y, cw_z)
        ccw_dev = (ccw_x, my_y, ccw_z)

        for c in range(K):
            send_chunk(x_ref, my_r, c, 0, 7, (my_x, dst_y, my_z))

        for c in range(K):
            wait_chunk(my_r, 7, c)
            send_chunk(out_ref, my_r, c, 1, 0, cw_dev)
            send_chunk(out_ref, my_r, c, 5, 4, ccw_dev)

        for s in range(1, 4):
            part_cw = (my_r - s) % N_RING
            part_ccw = (my_r + s) % N_RING
            for c in range(K):
                wait_chunk(part_cw, s - 1, c)
                send_chunk(out_ref, part_cw, c, 1 + s, s, cw_dev)
                if s <= 2:
                    wait_chunk(part_ccw, 4 + s - 1, c)
                    send_chunk(out_ref, part_ccw, c, 5 + s, 4 + s, ccw_dev)

        for c in range(K):
            wait_chunk((my_r - 4) % N_RING, 3, c)
            wait_chunk((my_r + 3) % N_RING, 6, c)

        for d in sends:
            d.wait_send()

    return pl.pallas_call(
        body,
        out_shape=jax.ShapeDtypeStruct(x.shape, jnp.float32),
        in_specs=[
            pl.BlockSpec(memory_space=pltpu.SMEM),
            pl.BlockSpec(memory_space=pl.ANY),
        ],
        out_specs=pl.BlockSpec(memory_space=pltpu.VMEM),
        scratch_shapes=[
            pltpu.SemaphoreType.DMA((8 * K,)),
            pltpu.SemaphoreType.DMA((8 * K,)),
        ],
        compiler_params=pltpu.CompilerParams(collective_id=0),
    )(pi, x)


# device time: 69068 ns/iter; 2.9068x vs baseline; 1.0901x over previous
import jax
import jax.numpy as jnp
from jax import lax
from jax.experimental import pallas as pl
from jax.experimental.pallas import tpu as pltpu

N_Y = 4
N_RING = 8
ROWS = 2048
PART = ROWS // N_RING
K = 8
CHUNK = PART // K


def kernel(x, pi):
    def body(pi_ref, x_ref, out_ref, send_sems, recv_sems):
        my_x = lax.axis_index("x")
        my_y = lax.axis_index("y")
        my_z = lax.axis_index("z")
        dst_y = pi_ref[my_y]
        src_y = jnp.int32(0)
        for j in range(N_Y):
            src_y = jnp.where(pi_ref[j] == my_y, jnp.int32(j), src_y)

        my_r = jnp.where(my_x == 0, my_z, 7 - my_z).astype(jnp.int32)

        def ring_coords(r):
            rx = jnp.where(r >= 4, 1, 0).astype(jnp.int32)
            rz = jnp.where(r >= 4, 7 - r, r).astype(jnp.int32)
            return rx, rz

        cw_x, cw_z = ring_coords((my_r + 1) % N_RING)
        ccw_x, ccw_z = ring_coords((my_r - 1) % N_RING)

        def blk(ref, p, c):
            return ref.at[:, pl.ds(p * PART + c * CHUNK, CHUNK), :]

        def wait_chunk(p, stage, c):
            pltpu.make_async_remote_copy(
                src_ref=blk(out_ref, p, c),
                dst_ref=blk(out_ref, p, c),
                send_sem=send_sems.at[0],
                recv_sem=recv_sems.at[stage * K + c],
                device_id=(0, 0, 0),
                device_id_type=pl.DeviceIdType.MESH,
            ).wait_recv()

        sends = []

        def send_chunk(src_ref, p, c, send_stage, recv_stage, dev):
            d = pltpu.make_async_remote_copy(
                src_ref=blk(src_ref, p, c),
                dst_ref=blk(out_ref, p, c),
                send_sem=send_sems.at[send_stage * K + c],
                recv_sem=recv_sems.at[recv_stage * K + c],
                device_id=dev,
                device_id_type=pl.DeviceIdType.MESH,
            )
            d.start()
            sends.append(d)

        barrier = pltpu.get_barrier_semaphore()
        for dev in [
            (my_x, src_y, my_z),
            (ccw_x, my_y, ccw_z),
            (cw_x, my_y, cw_z),
        ]:
            pl.semaphore_signal(
                barrier, inc=1, device_id=dev,
                device_id_type=pl.DeviceIdType.MESH,
            )
        pl.semaphore_wait(barrier, 3)

        cw_dev = (cw_x, my_y, cw_z)
        ccw_dev = (ccw_x, my_y, ccw_z)

        for c in range(K):
            send_chunk(x_ref, my_r, c, 0, 8, (my_x, dst_y, my_z))

        for c in range(K):
            wait_chunk(my_r, 8, c)
            send_chunk(out_ref, my_r, c, 1, 0, cw_dev)
            send_chunk(out_ref, my_r, c, 5, 4, ccw_dev)

        for s in range(1, 3):
            part_cw = (my_r - s) % N_RING
            part_ccw = (my_r + s) % N_RING
            for c in range(K):
                wait_chunk(part_cw, s - 1, c)
                send_chunk(out_ref, part_cw, c, 1 + s, s, cw_dev)
                wait_chunk(part_ccw, 4 + s - 1, c)
                send_chunk(out_ref, part_ccw, c, 5 + s, 4 + s, ccw_dev)

        part_cw = (my_r - 3) % N_RING
        part_ccw = (my_r + 3) % N_RING
        for c in range(K):
            wait_chunk(part_cw, 2, c)
            if c < K // 2:
                send_chunk(out_ref, part_cw, c, 4, 3, cw_dev)
            wait_chunk(part_ccw, 6, c)
            if c >= K // 2:
                send_chunk(out_ref, part_ccw, c, 8, 7, ccw_dev)

        for c in range(K // 2):
            wait_chunk((my_r + 4) % N_RING, 3, c)
        for c in range(K // 2, K):
            wait_chunk((my_r + 4) % N_RING, 7, c)

        for d in sends:
            d.wait_send()

    return pl.pallas_call(
        body,
        out_shape=jax.ShapeDtypeStruct(x.shape, jnp.float32),
        in_specs=[
            pl.BlockSpec(memory_space=pltpu.SMEM),
            pl.BlockSpec(memory_space=pl.ANY),
        ],
        out_specs=pl.BlockSpec(memory_space=pltpu.VMEM),
        scratch_shapes=[
            pltpu.SemaphoreType.DMA((9 * K,)),
            pltpu.SemaphoreType.DMA((9 * K,)),
        ],
        compiler_params=pltpu.CompilerParams(collective_id=0),
    )(pi, x)
